# baseline (device time: 112827 ns/iter reference)
import jax
import jax.numpy as jnp
from jax import lax
from jax.experimental import pallas as pl
from jax.experimental.pallas import tpu as pltpu

N_DEV = 8
B = 2
SQ = 128
D_MODEL = 512
HQ_LOCAL = 4
DH = 64
SKV_LOC = 128
SKV = SKV_LOC * N_DEV
BLK = 64


def kernel(x, Wq, K_ext, V_ext, Wo):
    def body(
        x_ref, wq_ref, k_ref, v_ref, wo_ref, out_ref,
        kbuf, vbuf, kgath, vgath, psend, pbuf,
        ksend_sems, krecv_sems, vsend_sems, vrecv_sems,
        psend_sems, precv_sems,
    ):
        my = lax.axis_index("i")

        barrier_sem = pltpu.get_barrier_semaphore()
        for d in range(1, N_DEV):
            peer = (my + d) % N_DEV
            pl.semaphore_signal(
                barrier_sem, inc=1, device_id=(peer,),
                device_id_type=pl.DeviceIdType.MESH,
            )
        pl.semaphore_wait(barrier_sem, N_DEV - 1)

        k_rdmas = []
        v_rdmas = []
        for d in range(1, N_DEV):
            tgt = (my + d) % N_DEV
            hstart = tgt * HQ_LOCAL
            kr = pltpu.make_async_remote_copy(
                src_ref=k_ref.at[:, :, pl.ds(hstart, HQ_LOCAL), :],
                dst_ref=kbuf.at[d - 1],
                send_sem=ksend_sems.at[d - 1],
                recv_sem=krecv_sems.at[d - 1],
                device_id=(tgt,),
                device_id_type=pl.DeviceIdType.MESH,
            )
            kr.start()
            k_rdmas.append(kr)
            vr = pltpu.make_async_remote_copy(
                src_ref=v_ref.at[:, :, pl.ds(hstart, HQ_LOCAL), :],
                dst_ref=vbuf.at[d - 1],
                send_sem=vsend_sems.at[d - 1],
                recv_sem=vrecv_sems.at[d - 1],
                device_id=(tgt,),
                device_id_type=pl.DeviceIdType.MESH,
            )
            vr.start()
            v_rdmas.append(vr)

        wq = wq_ref[:, :]
        qs = [
            jax.lax.dot(x_ref[b], wq, preferred_element_type=jnp.float32)
            for b in range(B)
        ]

        my_h = my * HQ_LOCAL
        my_off = my * SKV_LOC
        kgath[:, pl.ds(my_off, SKV_LOC), :, :] = k_ref[
            :, :, pl.ds(my_h, HQ_LOCAL), :
        ]
        vgath[:, pl.ds(my_off, SKV_LOC), :, :] = v_ref[
            :, :, pl.ds(my_h, HQ_LOCAL), :
        ]

        for d in range(1, N_DEV):
            src = (my - d) % N_DEV
            off = src * SKV_LOC
            k_rdmas[d - 1].wait_recv()
            kgath[:, pl.ds(off, SKV_LOC), :, :] = kbuf[d - 1]
            v_rdmas[d - 1].wait_recv()
            vgath[:, pl.ds(off, SKV_LOC), :, :] = vbuf[d - 1]

        qb = lax.broadcasted_iota(jnp.int32, (SQ, SKV), 0) // BLK
        kb = lax.broadcasted_iota(jnp.int32, (SQ, SKV), 1) // BLK
        mask = (qb == kb) | (kb == 0) | ((qb + kb) % 3 == 0)

        for b in range(B):
            ctxs = []
            for h in range(HQ_LOCAL):
                q_bh = qs[b][:, h * DH:(h + 1) * DH]
                k_bh = kgath[b, :, h, :]
                s = lax.dot_general(
                    q_bh, k_bh, (((1,), (1,)), ((), ())),
                    preferred_element_type=jnp.float32,
                ) * 0.125
                s = jnp.where(mask, s, -1e9)
                m = jnp.max(s, axis=1, keepdims=True)
                w = jnp.exp(s - m)
                w = w / jnp.sum(w, axis=1, keepdims=True)
                v_bh = vgath[b, :, h, :]
                ctxs.append(
                    jax.lax.dot(w, v_bh, preferred_element_type=jnp.float32)
                )
            ctx_b = jnp.concatenate(ctxs, axis=1)
            psend[pl.ds(b * SQ, SQ), :] = jax.lax.dot(
                ctx_b, wo_ref[:, :], preferred_element_type=jnp.float32
            )

        p_rdmas = []
        for d in range(1, N_DEV):
            tgt = (my + d) % N_DEV
            pr = pltpu.make_async_remote_copy(
                src_ref=psend,
                dst_ref=pbuf.at[d - 1],
                send_sem=psend_sems.at[d - 1],
                recv_sem=precv_sems.at[d - 1],
                device_id=(tgt,),
                device_id_type=pl.DeviceIdType.MESH,
            )
            pr.start()
            p_rdmas.append(pr)

        total = psend[:, :]
        for d in range(1, N_DEV):
            p_rdmas[d - 1].wait_recv()
            total = total + pbuf[d - 1]
        for b in range(B):
            out_ref[b, :, :] = total[b * SQ:(b + 1) * SQ, :]

        for d in range(1, N_DEV):
            k_rdmas[d - 1].wait_send()
            v_rdmas[d - 1].wait_send()
            p_rdmas[d - 1].wait_send()

    return pl.pallas_call(
        body,
        out_shape=jax.ShapeDtypeStruct((B, SQ, D_MODEL), jnp.float32),
        in_specs=[pl.BlockSpec(memory_space=pltpu.VMEM)] * 5,
        out_specs=pl.BlockSpec(memory_space=pltpu.VMEM),
        scratch_shapes=[
            pltpu.VMEM((N_DEV - 1, B, SKV_LOC, HQ_LOCAL, DH), jnp.float32),
            pltpu.VMEM((N_DEV - 1, B, SKV_LOC, HQ_LOCAL, DH), jnp.float32),
            pltpu.VMEM((B, SKV, HQ_LOCAL, DH), jnp.float32),
            pltpu.VMEM((B, SKV, HQ_LOCAL, DH), jnp.float32),
            pltpu.VMEM((B * SQ, D_MODEL), jnp.float32),
            pltpu.VMEM((N_DEV - 1, B * SQ, D_MODEL), jnp.float32),
            pltpu.SemaphoreType.DMA((N_DEV - 1,)),
            pltpu.SemaphoreType.DMA((N_DEV - 1,)),
            pltpu.SemaphoreType.DMA((N_DEV - 1,)),
            pltpu.SemaphoreType.DMA((N_DEV - 1,)),
            pltpu.SemaphoreType.DMA((N_DEV - 1,)),
            pltpu.SemaphoreType.DMA((N_DEV - 1,)),
        ],
        compiler_params=pltpu.CompilerParams(collective_id=0),
    )(x, Wq, K_ext, V_ext, Wo)


# device time: 107661 ns/iter; 1.0480x vs baseline; 1.0480x over previous
import jax
import jax.numpy as jnp
from jax import lax
from jax.experimental import pallas as pl
from jax.experimental.pallas import tpu as pltpu

N_DEV = 8
B = 2
SQ = 128
D_MODEL = 512
HQ_LOCAL = 4
DH = 64
SKV_LOC = 128
SKV = SKV_LOC * N_DEV
BLK = 64

QBLOCKS = {0: (0, 3, 6, 9, 12, 15), 1: (0, 1, 2, 5, 8, 11, 14)}
BFLY = (1, 3, 4)


def kernel(x, Wq, K_ext, V_ext, Wo):
    def body(
        x_ref, wq_ref, k_ref, v_ref, wo_ref, out_ref,
        kgath, vgath, pacc, pin,
        ksend_sems, krecv_sems, vsend_sems, vrecv_sems,
        psend_sems, precv_sems,
    ):
        my = lax.axis_index("i")

        barrier_sem = pltpu.get_barrier_semaphore()
        for d in range(1, N_DEV):
            peer = (my + d) % N_DEV
            pl.semaphore_signal(
                barrier_sem, inc=1, device_id=(peer,),
                device_id_type=pl.DeviceIdType.MESH,
            )
        pl.semaphore_wait(barrier_sem, N_DEV - 1)

        my_off = my * SKV_LOC
        k_rdmas = []
        v_rdmas = []
        for d in range(1, N_DEV):
            tgt = (my + d) % N_DEV
            hstart = tgt * HQ_LOCAL
            kr = pltpu.make_async_remote_copy(
                src_ref=k_ref.at[:, :, pl.ds(hstart, HQ_LOCAL), :],
                dst_ref=kgath.at[:, pl.ds(my_off, SKV_LOC), :, :],
                send_sem=ksend_sems.at[d - 1],
                recv_sem=krecv_sems.at[d - 1],
                device_id=(tgt,),
                device_id_type=pl.DeviceIdType.MESH,
            )
            kr.start()
            k_rdmas.append(kr)
            vr = pltpu.make_async_remote_copy(
                src_ref=v_ref.at[:, :, pl.ds(hstart, HQ_LOCAL), :],
                dst_ref=vgath.at[:, pl.ds(my_off, SKV_LOC), :, :],
                send_sem=vsend_sems.at[d - 1],
                recv_sem=vrecv_sems.at[d - 1],
                device_id=(tgt,),
                device_id_type=pl.DeviceIdType.MESH,
            )
            vr.start()
            v_rdmas.append(vr)

        wq = wq_ref[:, :]
        qs = [
            jax.lax.dot(x_ref[b], wq, preferred_element_type=jnp.float32)
            for b in range(B)
        ]

        my_h = my * HQ_LOCAL
        kgath[:, pl.ds(my_off, SKV_LOC), :, :] = k_ref[
            :, :, pl.ds(my_h, HQ_LOCAL), :
        ]
        vgath[:, pl.ds(my_off, SKV_LOC), :, :] = v_ref[
            :, :, pl.ds(my_h, HQ_LOCAL), :
        ]

        for d in range(1, N_DEV):
            k_rdmas[d - 1].wait_recv()
            v_rdmas[d - 1].wait_recv()

        wo = wo_ref[:, :]
        for b in range(B):
            for qb in range(2):
                blocks = QBLOCKS[qb]
                ctxs = []
                for h in range(HQ_LOCAL):
                    q_bh = qs[b][qb * BLK:(qb + 1) * BLK, h * DH:(h + 1) * DH]
                    k_sub = jnp.concatenate(
                        [kgath[b, kb * BLK:(kb + 1) * BLK, h, :] for kb in blocks],
                        axis=0,
                    )
                    s = lax.dot_general(
                        q_bh, k_sub, (((1,), (1,)), ((), ())),
                        preferred_element_type=jnp.float32,
                    ) * 0.125
                    m = jnp.max(s, axis=1, keepdims=True)
                    w = jnp.exp(s - m)
                    w = w / jnp.sum(w, axis=1, keepdims=True)
                    v_sub = jnp.concatenate(
                        [vgath[b, kb * BLK:(kb + 1) * BLK, h, :] for kb in blocks],
                        axis=0,
                    )
                    ctxs.append(
                        jax.lax.dot(w, v_sub, preferred_element_type=jnp.float32)
                    )
                ctx_row = jnp.concatenate(ctxs, axis=1)
                pacc[pl.ds(b * SQ + qb * BLK, BLK), :] = jax.lax.dot(
                    ctx_row, wo, preferred_element_type=jnp.float32
                )

        for step, dim in enumerate(BFLY):
            partner = lax.bitwise_xor(my, dim)
            pr = pltpu.make_async_remote_copy(
                src_ref=pacc,
                dst_ref=pin.at[step],
                send_sem=psend_sems.at[step],
                recv_sem=precv_sems.at[step],
                device_id=(partner,),
                device_id_type=pl.DeviceIdType.MESH,
            )
            pr.start()
            pr.wait_send()
            pr.wait_recv()
            pacc[:, :] = pacc[:, :] + pin[step]

        total = pacc[:, :]
        for b in range(B):
            out_ref[b, :, :] = total[b * SQ:(b + 1) * SQ, :]

        for d in range(1, N_DEV):
            k_rdmas[d - 1].wait_send()
            v_rdmas[d - 1].wait_send()

    return pl.pallas_call(
        body,
        out_shape=jax.ShapeDtypeStruct((B, SQ, D_MODEL), jnp.float32),
        in_specs=[pl.BlockSpec(memory_space=pltpu.VMEM)] * 5,
        out_specs=pl.BlockSpec(memory_space=pltpu.VMEM),
        scratch_shapes=[
            pltpu.VMEM((B, SKV, HQ_LOCAL, DH), jnp.float32),
            pltpu.VMEM((B, SKV, HQ_LOCAL, DH), jnp.float32),
            pltpu.VMEM((B * SQ, D_MODEL), jnp.float32),
            pltpu.VMEM((len(BFLY), B * SQ, D_MODEL), jnp.float32),
            pltpu.SemaphoreType.DMA((N_DEV - 1,)),
            pltpu.SemaphoreType.DMA((N_DEV - 1,)),
            pltpu.SemaphoreType.DMA((N_DEV - 1,)),
            pltpu.SemaphoreType.DMA((N_DEV - 1,)),
            pltpu.SemaphoreType.DMA((len(BFLY),)),
            pltpu.SemaphoreType.DMA((len(BFLY),)),
        ],
        compiler_params=pltpu.CompilerParams(collective_id=0),
    )(x, Wq, K_ext, V_ext, Wo)


# device time: 60153 ns/iter; 1.8757x vs baseline; 1.7898x over previous
import jax
import jax.numpy as jnp
from jax import lax
from jax.experimental import pallas as pl
from jax.experimental.pallas import tpu as pltpu

N_DEV = 8
B = 2
SQ = 128
D_MODEL = 512
HQ_LOCAL = 4
DH = 64
HD = HQ_LOCAL * DH
SKV_LOC = 128
SKV = SKV_LOC * N_DEV
BLK = 64

QBLOCKS = {0: (0, 3, 6, 9, 12, 15), 1: (0, 1, 2, 5, 8, 11, 14)}
BFLY = (1, 3, 4)


def kernel(x, Wq, K_ext, V_ext, Wo):
    k2 = K_ext.reshape(B, SKV_LOC, N_DEV * HD)
    v2 = V_ext.reshape(B, SKV_LOC, N_DEV * HD)

    def body(
        x_ref, wq_ref, k_ref, v_ref, wo_ref, out_ref,
        kgath, vgath, pacc, pin,
        ksend_sems, krecv_sems, vsend_sems, vrecv_sems,
        psend_sems, precv_sems,
    ):
        my = lax.axis_index("i")

        barrier_sem = pltpu.get_barrier_semaphore()
        for d in range(1, N_DEV):
            peer = (my + d) % N_DEV
            pl.semaphore_signal(
                barrier_sem, inc=1, device_id=(peer,),
                device_id_type=pl.DeviceIdType.MESH,
            )
        pl.semaphore_wait(barrier_sem, N_DEV - 1)

        my_off = my * SKV_LOC
        k_rdmas = []
        v_rdmas = []
        for d in range(1, N_DEV):
            tgt = (my + d) % N_DEV
            lstart = tgt * HD
            kr = pltpu.make_async_remote_copy(
                src_ref=k_ref.at[:, :, pl.ds(lstart, HD)],
                dst_ref=kgath.at[:, pl.ds(my_off, SKV_LOC), :],
                send_sem=ksend_sems.at[d - 1],
                recv_sem=krecv_sems.at[d - 1],
                device_id=(tgt,),
                device_id_type=pl.DeviceIdType.MESH,
            )
            kr.start()
            k_rdmas.append(kr)
            vr = pltpu.make_async_remote_copy(
                src_ref=v_ref.at[:, :, pl.ds(lstart, HD)],
                dst_ref=vgath.at[:, pl.ds(my_off, SKV_LOC), :],
                send_sem=vsend_sems.at[d - 1],
                recv_sem=vrecv_sems.at[d - 1],
                device_id=(tgt,),
                device_id_type=pl.DeviceIdType.MESH,
            )
            vr.start()
            v_rdmas.append(vr)

        wq = wq_ref[:, :]
        qs = [
            jax.lax.dot(x_ref[b], wq, preferred_element_type=jnp.float32)
            for b in range(B)
        ]

        my_l = my * HD
        kgath[:, pl.ds(my_off, SKV_LOC), :] = k_ref[:, :, pl.ds(my_l, HD)]
        vgath[:, pl.ds(my_off, SKV_LOC), :] = v_ref[:, :, pl.ds(my_l, HD)]

        for d in range(1, N_DEV):
            k_rdmas[d - 1].wait_recv()
            v_rdmas[d - 1].wait_recv()

        wo = wo_ref[:, :]

        def attend(b):
            for qb in range(2):
                blocks = QBLOCKS[qb]
                k_rows = jnp.concatenate(
                    [kgath[b, kb * BLK:(kb + 1) * BLK, :] for kb in blocks],
                    axis=0,
                )
                v_rows = jnp.concatenate(
                    [vgath[b, kb * BLK:(kb + 1) * BLK, :] for kb in blocks],
                    axis=0,
                )
                ctxs = []
                for h in range(HQ_LOCAL):
                    q_bh = qs[b][qb * BLK:(qb + 1) * BLK, h * DH:(h + 1) * DH]
                    k_sub = k_rows[:, h * DH:(h + 1) * DH]
                    s = lax.dot_general(
                        q_bh, k_sub, (((1,), (1,)), ((), ())),
                        preferred_element_type=jnp.float32,
                    ) * 0.125
                    m = jnp.max(s, axis=1, keepdims=True)
                    w = jnp.exp(s - m)
                    w = w / jnp.sum(w, axis=1, keepdims=True)
                    ctxs.append(
                        jax.lax.dot(
                            w, v_rows[:, h * DH:(h + 1) * DH],
                            preferred_element_type=jnp.float32,
                        )
                    )
                ctx_row = jnp.concatenate(ctxs, axis=1)
                pacc[pl.ds(b * SQ + qb * BLK, BLK), :] = jax.lax.dot(
                    ctx_row, wo, preferred_element_type=jnp.float32
                )

        def bfly_start(step, half):
            partner = lax.bitwise_xor(my, BFLY[step])
            pr = pltpu.make_async_remote_copy(
                src_ref=pacc.at[pl.ds(half * SQ, SQ), :],
                dst_ref=pin.at[step, pl.ds(half * SQ, SQ), :],
                send_sem=psend_sems.at[step, half],
                recv_sem=precv_sems.at[step, half],
                device_id=(partner,),
                device_id_type=pl.DeviceIdType.MESH,
            )
            pr.start()
            return pr

        def bfly_finish(pr, step, half):
            pr.wait_send()
            pr.wait_recv()
            pacc[pl.ds(half * SQ, SQ), :] = (
                pacc[pl.ds(half * SQ, SQ), :] + pin[step, pl.ds(half * SQ, SQ), :]
            )

        attend(0)
        pr00 = bfly_start(0, 0)
        attend(1)
        pr01 = bfly_start(0, 1)
        bfly_finish(pr00, 0, 0)
        pr10 = bfly_start(1, 0)
        bfly_finish(pr01, 0, 1)
        pr11 = bfly_start(1, 1)
        bfly_finish(pr10, 1, 0)
        pr20 = bfly_start(2, 0)
        bfly_finish(pr11, 1, 1)
        pr21 = bfly_start(2, 1)
        bfly_finish(pr20, 2, 0)
        out_ref[0, :, :] = pacc[pl.ds(0, SQ), :]
        bfly_finish(pr21, 2, 1)
        out_ref[1, :, :] = pacc[pl.ds(SQ, SQ), :]

        for d in range(1, N_DEV):
            k_rdmas[d - 1].wait_send()
            v_rdmas[d - 1].wait_send()

    return pl.pallas_call(
        body,
        out_shape=jax.ShapeDtypeStruct((B, SQ, D_MODEL), jnp.float32),
        in_specs=[pl.BlockSpec(memory_space=pltpu.VMEM)] * 5,
        out_specs=pl.BlockSpec(memory_space=pltpu.VMEM),
        scratch_shapes=[
            pltpu.VMEM((B, SKV, HD), jnp.float32),
            pltpu.VMEM((B, SKV, HD), jnp.float32),
            pltpu.VMEM((B * SQ, D_MODEL), jnp.float32),
            pltpu.VMEM((len(BFLY), B * SQ, D_MODEL), jnp.float32),
            pltpu.SemaphoreType.DMA((N_DEV - 1,)),
            pltpu.SemaphoreType.DMA((N_DEV - 1,)),
            pltpu.SemaphoreType.DMA((N_DEV - 1,)),
            pltpu.SemaphoreType.DMA((N_DEV - 1,)),
            pltpu.SemaphoreType.DMA((len(BFLY), B)),
            pltpu.SemaphoreType.DMA((len(BFLY), B)),
        ],
        compiler_params=pltpu.CompilerParams(collective_id=0),
    )(x, Wq, k2, v2, Wo)


# device time: 45130 ns/iter; 2.5000x vs baseline; 1.3329x over previous
import jax
import jax.numpy as jnp
from jax import lax
from jax.experimental import pallas as pl
from jax.experimental.pallas import tpu as pltpu

N_DEV = 8
B = 2
SQ = 128
D_MODEL = 512
HQ_LOCAL = 4
DH = 64
HD = HQ_LOCAL * DH
SKV_LOC = 128
SKV = SKV_LOC * N_DEV
BLK = 64

QBLOCKS = {0: (0, 3, 6, 9, 12, 15), 1: (0, 1, 2, 5, 8, 11, 14)}
BFLY = (1, 3, 4)


def kernel(x, Wq, K_ext, V_ext, Wo):
    k2 = K_ext.reshape(B, SKV_LOC, N_DEV * HD).astype(jnp.bfloat16)
    v2 = V_ext.reshape(B, SKV_LOC, N_DEV * HD).astype(jnp.bfloat16)

    def body(
        x_ref, wq_ref, k_ref, v_ref, wo_ref, out_ref,
        kgath, vgath, pacc, pin,
        ksend_sems, krecv_sems, vsend_sems, vrecv_sems,
        psend_sems, precv_sems,
    ):
        my = lax.axis_index("i")

        barrier_sem = pltpu.get_barrier_semaphore()
        for d in range(1, N_DEV):
            peer = (my + d) % N_DEV
            pl.semaphore_signal(
                barrier_sem, inc=1, device_id=(peer,),
                device_id_type=pl.DeviceIdType.MESH,
            )
        pl.semaphore_wait(barrier_sem, N_DEV - 1)

        my_off = my * SKV_LOC
        k_rdmas = []
        v_rdmas = []
        for d in range(1, N_DEV):
            tgt = (my + d) % N_DEV
            lstart = tgt * HD
            kr = pltpu.make_async_remote_copy(
                src_ref=k_ref.at[:, :, pl.ds(lstart, HD)],
                dst_ref=kgath.at[:, pl.ds(my_off, SKV_LOC), :],
                send_sem=ksend_sems.at[d - 1],
                recv_sem=krecv_sems.at[d - 1],
                device_id=(tgt,),
                device_id_type=pl.DeviceIdType.MESH,
            )
            kr.start()
            k_rdmas.append(kr)
            vr = pltpu.make_async_remote_copy(
                src_ref=v_ref.at[:, :, pl.ds(lstart, HD)],
                dst_ref=vgath.at[:, pl.ds(my_off, SKV_LOC), :],
                send_sem=vsend_sems.at[d - 1],
                recv_sem=vrecv_sems.at[d - 1],
                device_id=(tgt,),
                device_id_type=pl.DeviceIdType.MESH,
            )
            vr.start()
            v_rdmas.append(vr)

        wq = wq_ref[:, :]
        qs = [
            jax.lax.dot(x_ref[b], wq, preferred_element_type=jnp.float32)
            for b in range(B)
        ]

        my_l = my * HD
        kgath[:, pl.ds(my_off, SKV_LOC), :] = k_ref[:, :, pl.ds(my_l, HD)]
        vgath[:, pl.ds(my_off, SKV_LOC), :] = v_ref[:, :, pl.ds(my_l, HD)]

        for d in range(1, N_DEV):
            k_rdmas[d - 1].wait_recv()
            v_rdmas[d - 1].wait_recv()

        wo = wo_ref[:, :]

        def attend(b):
            for qb in range(2):
                blocks = QBLOCKS[qb]
                k_rows = jnp.concatenate(
                    [kgath[b, kb * BLK:(kb + 1) * BLK, :] for kb in blocks],
                    axis=0,
                )
                v_rows = jnp.concatenate(
                    [vgath[b, kb * BLK:(kb + 1) * BLK, :] for kb in blocks],
                    axis=0,
                )
                ctxs = []
                for h in range(HQ_LOCAL):
                    q_bh = qs[b][qb * BLK:(qb + 1) * BLK, h * DH:(h + 1) * DH]
                    k_sub = k_rows[:, h * DH:(h + 1) * DH]
                    s = lax.dot_general(
                        q_bh.astype(jnp.bfloat16), k_sub,
                        (((1,), (1,)), ((), ())),
                        preferred_element_type=jnp.float32,
                    ) * 0.125
                    m = jnp.max(s, axis=1, keepdims=True)
                    w = jnp.exp(s - m)
                    w = w / jnp.sum(w, axis=1, keepdims=True)
                    ctxs.append(
                        jax.lax.dot(
                            w.astype(jnp.bfloat16), v_rows[:, h * DH:(h + 1) * DH],
                            preferred_element_type=jnp.float32,
                        )
                    )
                ctx_row = jnp.concatenate(ctxs, axis=1)
                pacc[pl.ds(b * SQ + qb * BLK, BLK), :] = jax.lax.dot(
                    ctx_row, wo, preferred_element_type=jnp.float32
                )

        def bfly_start(step, half):
            partner = lax.bitwise_xor(my, BFLY[step])
            pr = pltpu.make_async_remote_copy(
                src_ref=pacc.at[pl.ds(half * SQ, SQ), :],
                dst_ref=pin.at[step, pl.ds(half * SQ, SQ), :],
                send_sem=psend_sems.at[step, half],
                recv_sem=precv_sems.at[step, half],
                device_id=(partner,),
                device_id_type=pl.DeviceIdType.MESH,
            )
            pr.start()
            return pr

        def bfly_finish(pr, step, half):
            pr.wait_send()
            pr.wait_recv()
            pacc[pl.ds(half * SQ, SQ), :] = (
                pacc[pl.ds(half * SQ, SQ), :] + pin[step, pl.ds(half * SQ, SQ), :]
            )

        attend(0)
        pr00 = bfly_start(0, 0)
        attend(1)
        pr01 = bfly_start(0, 1)
        bfly_finish(pr00, 0, 0)
        pr10 = bfly_start(1, 0)
        bfly_finish(pr01, 0, 1)
        pr11 = bfly_start(1, 1)
        bfly_finish(pr10, 1, 0)
        pr20 = bfly_start(2, 0)
        bfly_finish(pr11, 1, 1)
        pr21 = bfly_start(2, 1)
        bfly_finish(pr20, 2, 0)
        out_ref[0, :, :] = pacc[pl.ds(0, SQ), :]
        bfly_finish(pr21, 2, 1)
        out_ref[1, :, :] = pacc[pl.ds(SQ, SQ), :]

        for d in range(1, N_DEV):
            k_rdmas[d - 1].wait_send()
            v_rdmas[d - 1].wait_send()

    return pl.pallas_call(
        body,
        out_shape=jax.ShapeDtypeStruct((B, SQ, D_MODEL), jnp.float32),
        in_specs=[pl.BlockSpec(memory_space=pltpu.VMEM)] * 5,
        out_specs=pl.BlockSpec(memory_space=pltpu.VMEM),
        scratch_shapes=[
            pltpu.VMEM((B, SKV, HD), jnp.bfloat16),
            pltpu.VMEM((B, SKV, HD), jnp.bfloat16),
            pltpu.VMEM((B * SQ, D_MODEL), jnp.float32),
            pltpu.VMEM((len(BFLY), B * SQ, D_MODEL), jnp.float32),
            pltpu.SemaphoreType.DMA((N_DEV - 1,)),
            pltpu.SemaphoreType.DMA((N_DEV - 1,)),
            pltpu.SemaphoreType.DMA((N_DEV - 1,)),
            pltpu.SemaphoreType.DMA((N_DEV - 1,)),
            pltpu.SemaphoreType.DMA((len(BFLY), B)),
            pltpu.SemaphoreType.DMA((len(BFLY), B)),
        ],
        compiler_params=pltpu.CompilerParams(collective_id=0),
    )(x, Wq, k2, v2, Wo)


# device time: 41089 ns/iter; 2.7459x vs baseline; 1.0983x over previous
import jax
import jax.numpy as jnp
from jax import lax
from jax.experimental import pallas as pl
from jax.experimental.pallas import tpu as pltpu

N_DEV = 8
B = 2
SQ = 128
D_MODEL = 512
HQ_LOCAL = 4
DH = 64
HD = HQ_LOCAL * DH
SKV_LOC = 128
SKV = SKV_LOC * N_DEV
BLK = 64

QBLOCKS = {0: (0, 3, 6, 9, 12, 15), 1: (0, 1, 2, 5, 8, 11, 14)}
BFLY = (1, 3, 4)


def kernel(x, Wq, K_ext, V_ext, Wo):
    k2 = K_ext.reshape(B, SKV_LOC, N_DEV * HD).astype(jnp.bfloat16)
    v2 = V_ext.reshape(B, SKV_LOC, N_DEV * HD).astype(jnp.bfloat16)
    x16 = x.astype(jnp.bfloat16)
    wq16 = Wq.astype(jnp.bfloat16)
    wo16 = Wo.astype(jnp.bfloat16)

    def body(
        x_ref, wq_ref, k_ref, v_ref, wo_ref, out_ref,
        kgath, vgath, pacc, pstage, pin,
        ksend_sems, krecv_sems, vsend_sems, vrecv_sems,
        psend_sems, precv_sems,
    ):
        my = lax.axis_index("i")

        barrier_sem = pltpu.get_barrier_semaphore()
        for d in range(1, N_DEV):
            peer = (my + d) % N_DEV
            pl.semaphore_signal(
                barrier_sem, inc=1, device_id=(peer,),
                device_id_type=pl.DeviceIdType.MESH,
            )
        pl.semaphore_wait(barrier_sem, N_DEV - 1)

        my_off = my * SKV_LOC
        k_rdmas = []
        v_rdmas = []
        for d in range(1, N_DEV):
            tgt = (my + d) % N_DEV
            lstart = tgt * HD
            kr = pltpu.make_async_remote_copy(
                src_ref=k_ref.at[:, :, pl.ds(lstart, HD)],
                dst_ref=kgath.at[:, pl.ds(my_off, SKV_LOC), :],
                send_sem=ksend_sems.at[d - 1],
                recv_sem=krecv_sems.at[d - 1],
                device_id=(tgt,),
                device_id_type=pl.DeviceIdType.MESH,
            )
            kr.start()
            k_rdmas.append(kr)
            vr = pltpu.make_async_remote_copy(
                src_ref=v_ref.at[:, :, pl.ds(lstart, HD)],
                dst_ref=vgath.at[:, pl.ds(my_off, SKV_LOC), :],
                send_sem=vsend_sems.at[d - 1],
                recv_sem=vrecv_sems.at[d - 1],
                device_id=(tgt,),
                device_id_type=pl.DeviceIdType.MESH,
            )
            vr.start()
            v_rdmas.append(vr)

        wq = wq_ref[:, :]
        qs = [
            jax.lax.dot(
                x_ref[b], wq, preferred_element_type=jnp.float32
            ).astype(jnp.bfloat16)
            for b in range(B)
        ]

        my_l = my * HD
        kgath[:, pl.ds(my_off, SKV_LOC), :] = k_ref[:, :, pl.ds(my_l, HD)]
        vgath[:, pl.ds(my_off, SKV_LOC), :] = v_ref[:, :, pl.ds(my_l, HD)]

        for d in range(1, N_DEV):
            k_rdmas[d - 1].wait_recv()
            v_rdmas[d - 1].wait_recv()

        wo = wo_ref[:, :]

        def attend(b):
            for qb in range(2):
                blocks = QBLOCKS[qb]
                k_rows = jnp.concatenate(
                    [kgath[b, kb * BLK:(kb + 1) * BLK, :] for kb in blocks],
                    axis=0,
                )
                v_rows = jnp.concatenate(
                    [vgath[b, kb * BLK:(kb + 1) * BLK, :] for kb in blocks],
                    axis=0,
                )
                ctxs = []
                for h in range(HQ_LOCAL):
                    q_bh = qs[b][qb * BLK:(qb + 1) * BLK, h * DH:(h + 1) * DH]
                    k_sub = k_rows[:, h * DH:(h + 1) * DH]
                    s = lax.dot_general(
                        q_bh, k_sub, (((1,), (1,)), ((), ())),
                        preferred_element_type=jnp.float32,
                    ) * 0.125
                    m = jnp.max(s, axis=1, keepdims=True)
                    w = jnp.exp(s - m)
                    w = w / jnp.sum(w, axis=1, keepdims=True)
                    ctxs.append(
                        jax.lax.dot(
                            w.astype(jnp.bfloat16), v_rows[:, h * DH:(h + 1) * DH],
                            preferred_element_type=jnp.float32,
                        )
                    )
                ctx_row = jnp.concatenate(ctxs, axis=1).astype(
                    jnp.bfloat16
                )
                pacc[pl.ds(b * SQ + qb * BLK, BLK), :] = jax.lax.dot(
                    ctx_row, wo, preferred_element_type=jnp.float32
                )

        def bfly_start(step, half):
            partner = lax.bitwise_xor(my, BFLY[step])
            pstage[step, pl.ds(half * SQ, SQ), :] = pacc[
                pl.ds(half * SQ, SQ), :
            ].astype(jnp.bfloat16)
            pr = pltpu.make_async_remote_copy(
                src_ref=pstage.at[step, pl.ds(half * SQ, SQ), :],
                dst_ref=pin.at[step, pl.ds(half * SQ, SQ), :],
                send_sem=psend_sems.at[step, half],
                recv_sem=precv_sems.at[step, half],
                device_id=(partner,),
                device_id_type=pl.DeviceIdType.MESH,
            )
            pr.start()
            return pr

        def bfly_finish(pr, step, half):
            pr.wait_send()
            pr.wait_recv()
            pacc[pl.ds(half * SQ, SQ), :] = (
                pacc[pl.ds(half * SQ, SQ), :] + pin[step, pl.ds(half * SQ, SQ), :]
            )

        attend(0)
        pr00 = bfly_start(0, 0)
        attend(1)
        pr01 = bfly_start(0, 1)
        bfly_finish(pr00, 0, 0)
        pr10 = bfly_start(1, 0)
        bfly_finish(pr01, 0, 1)
        pr11 = bfly_start(1, 1)
        bfly_finish(pr10, 1, 0)
        pr20 = bfly_start(2, 0)
        bfly_finish(pr11, 1, 1)
        pr21 = bfly_start(2, 1)
        bfly_finish(pr20, 2, 0)
        out_ref[0, :, :] = pacc[pl.ds(0, SQ), :]
        bfly_finish(pr21, 2, 1)
        out_ref[1, :, :] = pacc[pl.ds(SQ, SQ), :]

        for d in range(1, N_DEV):
            k_rdmas[d - 1].wait_send()
            v_rdmas[d - 1].wait_send()

    return pl.pallas_call(
        body,
        out_shape=jax.ShapeDtypeStruct((B, SQ, D_MODEL), jnp.float32),
        in_specs=[pl.BlockSpec(memory_space=pltpu.VMEM)] * 5,
        out_specs=pl.BlockSpec(memory_space=pltpu.VMEM),
        scratch_shapes=[
            pltpu.VMEM((B, SKV, HD), jnp.bfloat16),
            pltpu.VMEM((B, SKV, HD), jnp.bfloat16),
            pltpu.VMEM((B * SQ, D_MODEL), jnp.float32),
            pltpu.VMEM((len(BFLY), B * SQ, D_MODEL), jnp.bfloat16),
            pltpu.VMEM((len(BFLY), B * SQ, D_MODEL), jnp.bfloat16),
            pltpu.SemaphoreType.DMA((N_DEV - 1,)),
            pltpu.SemaphoreType.DMA((N_DEV - 1,)),
            pltpu.SemaphoreType.DMA((N_DEV - 1,)),
            pltpu.SemaphoreType.DMA((N_DEV - 1,)),
            pltpu.SemaphoreType.DMA((len(BFLY), B)),
            pltpu.SemaphoreType.DMA((len(BFLY), B)),
        ],
        compiler_params=pltpu.CompilerParams(collective_id=0),
    )(x16, wq16, k2, v2, wo16)


# device time: 32620 ns/iter; 3.4588x vs baseline; 1.2596x over previous
import jax
import jax.numpy as jnp
from jax import lax
from jax.experimental import pallas as pl
from jax.experimental.pallas import tpu as pltpu

N_DEV = 8
B = 2
SQ = 128
D_MODEL = 512
HQ_LOCAL = 4
DH = 64
HD = HQ_LOCAL * DH
SKV_LOC = 128
SKV = SKV_LOC * N_DEV
BLK = 64

QBLOCKS = {0: (0, 3, 6, 9, 12, 15), 1: (0, 1, 2, 5, 8, 11, 14)}
BFLY = (1, 3, 4)


def kernel(x, Wq, K_ext, V_ext, Wo):
    k2 = K_ext.reshape(B, SKV_LOC, N_DEV * HD).astype(jnp.bfloat16)
    v2 = V_ext.reshape(B, SKV_LOC, N_DEV * HD).astype(jnp.bfloat16)
    x16 = x.astype(jnp.bfloat16)
    wq16 = Wq.astype(jnp.bfloat16)
    wo16 = Wo.astype(jnp.bfloat16)

    def body(
        x_ref, wq_ref, k_ref, v_ref, wo_ref, out_ref,
        kgath, vgath, pacc, pstage, pin,
        ksend_sems, krecv_sems, vsend_sems, vrecv_sems,
        psend_sems, precv_sems,
    ):
        my = lax.axis_index("i")

        barrier_sem = pltpu.get_barrier_semaphore()
        for d in range(1, N_DEV):
            peer = (my + d) % N_DEV
            pl.semaphore_signal(
                barrier_sem, inc=1, device_id=(peer,),
                device_id_type=pl.DeviceIdType.MESH,
            )
        pl.semaphore_wait(barrier_sem, N_DEV - 1)

        my_off = my * SKV_LOC
        k_rdmas = []
        v_rdmas = []
        for d in range(1, N_DEV):
            tgt = (my + d) % N_DEV
            lstart = tgt * HD
            kr = pltpu.make_async_remote_copy(
                src_ref=k_ref.at[:, :, pl.ds(lstart, HD)],
                dst_ref=kgath.at[:, pl.ds(my_off, SKV_LOC), :],
                send_sem=ksend_sems.at[d - 1],
                recv_sem=krecv_sems.at[d - 1],
                device_id=(tgt,),
                device_id_type=pl.DeviceIdType.MESH,
            )
            kr.start()
            k_rdmas.append(kr)
        for d in range(1, N_DEV):
            tgt = (my + d) % N_DEV
            lstart = tgt * HD
            vr = pltpu.make_async_remote_copy(
                src_ref=v_ref.at[:, :, pl.ds(lstart, HD)],
                dst_ref=vgath.at[:, pl.ds(my_off, SKV_LOC), :],
                send_sem=vsend_sems.at[d - 1],
                recv_sem=vrecv_sems.at[d - 1],
                device_id=(tgt,),
                device_id_type=pl.DeviceIdType.MESH,
            )
            vr.start()
            v_rdmas.append(vr)

        wq = wq_ref[:, :]
        qs = [
            jax.lax.dot(
                x_ref[b], wq, preferred_element_type=jnp.float32
            ).astype(jnp.bfloat16)
            for b in range(B)
        ]

        my_l = my * HD
        kgath[:, pl.ds(my_off, SKV_LOC), :] = k_ref[:, :, pl.ds(my_l, HD)]
        vgath[:, pl.ds(my_off, SKV_LOC), :] = v_ref[:, :, pl.ds(my_l, HD)]

        for d in range(1, N_DEV):
            k_rdmas[d - 1].wait_recv()

        weights = {}
        for b in range(B):
            for qb in range(2):
                blocks = QBLOCKS[qb]
                k_rows = jnp.concatenate(
                    [kgath[b, kb * BLK:(kb + 1) * BLK, :] for kb in blocks],
                    axis=0,
                )
                ws = []
                for h in range(HQ_LOCAL):
                    q_bh = qs[b][qb * BLK:(qb + 1) * BLK, h * DH:(h + 1) * DH]
                    k_sub = k_rows[:, h * DH:(h + 1) * DH]
                    s = lax.dot_general(
                        q_bh, k_sub, (((1,), (1,)), ((), ())),
                        preferred_element_type=jnp.float32,
                    ) * 0.125
                    m = jnp.max(s, axis=1, keepdims=True)
                    w = jnp.exp(s - m)
                    w = w / jnp.sum(w, axis=1, keepdims=True)
                    ws.append(w.astype(jnp.bfloat16))
                weights[(b, qb)] = ws

        for d in range(1, N_DEV):
            v_rdmas[d - 1].wait_recv()

        wo = wo_ref[:, :]

        def attend(b):
            for qb in range(2):
                blocks = QBLOCKS[qb]
                v_rows = jnp.concatenate(
                    [vgath[b, kb * BLK:(kb + 1) * BLK, :] for kb in blocks],
                    axis=0,
                )
                ctxs = [
                    jax.lax.dot(
                        weights[(b, qb)][h], v_rows[:, h * DH:(h + 1) * DH],
                        preferred_element_type=jnp.float32,
                    )
                    for h in range(HQ_LOCAL)
                ]
                ctx_row = jnp.concatenate(ctxs, axis=1).astype(
                    jnp.bfloat16
                )
                pacc[pl.ds(b * SQ + qb * BLK, BLK), :] = jax.lax.dot(
                    ctx_row, wo, preferred_element_type=jnp.float32
                )

        def bfly_start(step, half):
            partner = lax.bitwise_xor(my, BFLY[step])
            pstage[step, pl.ds(half * SQ, SQ), :] = pacc[
                pl.ds(half * SQ, SQ), :
            ].astype(jnp.bfloat16)
            pr = pltpu.make_async_remote_copy(
                src_ref=pstage.at[step, pl.ds(half * SQ, SQ), :],
                dst_ref=pin.at[step, pl.ds(half * SQ, SQ), :],
                send_sem=psend_sems.at[step, half],
                recv_sem=precv_sems.at[step, half],
                device_id=(partner,),
                device_id_type=pl.DeviceIdType.MESH,
            )
            pr.start()
            return pr

        def bfly_finish(pr, step, half):
            pr.wait_send()
            pr.wait_recv()
            pacc[pl.ds(half * SQ, SQ), :] = (
                pacc[pl.ds(half * SQ, SQ), :] + pin[step, pl.ds(half * SQ, SQ), :]
            )

        attend(0)
        pr00 = bfly_start(0, 0)
        attend(1)
        pr01 = bfly_start(0, 1)
        bfly_finish(pr00, 0, 0)
        pr10 = bfly_start(1, 0)
        bfly_finish(pr01, 0, 1)
        pr11 = bfly_start(1, 1)
        bfly_finish(pr10, 1, 0)
        pr20 = bfly_start(2, 0)
        bfly_finish(pr11, 1, 1)
        pr21 = bfly_start(2, 1)
        bfly_finish(pr20, 2, 0)
        out_ref[0, :, :] = pacc[pl.ds(0, SQ), :]
        bfly_finish(pr21, 2, 1)
        out_ref[1, :, :] = pacc[pl.ds(SQ, SQ), :]

        for d in range(1, N_DEV):
            k_rdmas[d - 1].wait_send()
            v_rdmas[d - 1].wait_send()

    return pl.pallas_call(
        body,
        out_shape=jax.ShapeDtypeStruct((B, SQ, D_MODEL), jnp.float32),
        in_specs=[pl.BlockSpec(memory_space=pltpu.VMEM)] * 5,
        out_specs=pl.BlockSpec(memory_space=pltpu.VMEM),
        scratch_shapes=[
            pltpu.VMEM((B, SKV, HD), jnp.bfloat16),
            pltpu.VMEM((B, SKV, HD), jnp.bfloat16),
            pltpu.VMEM((B * SQ, D_MODEL), jnp.float32),
            pltpu.VMEM((len(BFLY), B * SQ, D_MODEL), jnp.bfloat16),
            pltpu.VMEM((len(BFLY), B * SQ, D_MODEL), jnp.bfloat16),
            pltpu.SemaphoreType.DMA((N_DEV - 1,)),
            pltpu.SemaphoreType.DMA((N_DEV - 1,)),
            pltpu.SemaphoreType.DMA((N_DEV - 1,)),
            pltpu.SemaphoreType.DMA((N_DEV - 1,)),
            pltpu.SemaphoreType.DMA((len(BFLY), B)),
            pltpu.SemaphoreType.DMA((len(BFLY), B)),
        ],
        compiler_params=pltpu.CompilerParams(collective_id=0),
    )(x16, wq16, k2, v2, wo16)
